# baseline (speedup 1.0000x reference)
import functools

import jax
import jax.numpy as jnp
from jax.experimental import pallas as pl
from jax.experimental.pallas import tpu as pltpu

_PREC = jax.lax.Precision.HIGHEST
_DN = (((1,), (0,)), ((), ()))


def _proj_kernel(x_ref, w1_ref, w4_ref, o1_ref, o4_ref):
    x = x_ref[...].astype(jnp.bfloat16)
    o1_ref[...] = jax.lax.dot_general(
        x, w1_ref[...].astype(jnp.bfloat16), _DN,
        preferred_element_type=jnp.float32).astype(jnp.bfloat16)
    o4_ref[...] = jax.lax.dot_general(
        x, w4_ref[...].astype(jnp.bfloat16), _DN,
        preferred_element_type=jnp.float32).astype(jnp.bfloat16)


def _initial_supports(x, w1, w4):
    n, _ = x.shape
    f1, f4 = w1.shape[1], w4.shape[1]
    return pl.pallas_call(
        _proj_kernel,
        out_shape=(jax.ShapeDtypeStruct((n, f1), jnp.bfloat16),
                   jax.ShapeDtypeStruct((n, f4), jnp.bfloat16)),
    )(x, w1, w4)


def _epilogue(h, b_ref, w_ref, c_ref, out_dtype):
    h = jnp.maximum(h + b_ref[...], 0.0).astype(jnp.bfloat16)
    o = jax.lax.dot_general(
        h, w_ref[...].astype(jnp.bfloat16), _DN,
        preferred_element_type=jnp.float32) + c_ref[...]
    return o.astype(out_dtype)


def _agg_first_kernel(a_ref, s_ref, b_ref, w_ref, c_ref, o_ref, abf_ref):
    a = a_ref[...].astype(jnp.bfloat16)
    k = a_ref.shape[1]
    pad = abf_ref.shape[1] - k
    if pad:
        a_pad = jnp.concatenate(
            [a, jnp.zeros((a.shape[0], pad), jnp.bfloat16)], axis=1)
    else:
        a_pad = a
    abf_ref[...] = a_pad
    h = jax.lax.dot_general(
        a, s_ref[...], _DN, preferred_element_type=jnp.float32)
    o_ref[...] = _epilogue(h, b_ref, w_ref, c_ref, o_ref.dtype)


def _agg_rest_kernel(a_ref, st_ref, b_ref, w_ref, c_ref, o_ref, *, nvalid):
    ht = jax.lax.dot_general(
        st_ref[...], a_ref[...], (((1,), (1,)), ((), ())),
        preferred_element_type=jnp.float32)
    ht = jnp.maximum(ht + b_ref[...], 0.0).astype(jnp.bfloat16)
    ot = jax.lax.dot_general(
        w_ref[...].astype(jnp.bfloat16), ht, (((0,), (0,)), ((), ())),
        preferred_element_type=jnp.float32) + c_ref[...]
    g, bm = o_ref.shape
    col = pl.program_id(0) * bm + jax.lax.broadcasted_iota(jnp.int32, (g, bm), 1)
    ot = jnp.where(col < nvalid, ot, 0.0)
    o_ref[...] = ot.astype(o_ref.dtype)


def _agg_first(adj, s, b, w, c, bm):
    n = adj.shape[0]
    npad = -(-n // 256) * 256
    f = s.shape[1]
    g = w.shape[1]
    return pl.pallas_call(
        _agg_first_kernel,
        grid=(n // bm,),
        in_specs=[
            pl.BlockSpec((bm, n), lambda i: (i, 0)),
            pl.BlockSpec((n, f), lambda i: (0, 0)),
            pl.BlockSpec((1, f), lambda i: (0, 0)),
            pl.BlockSpec((f, g), lambda i: (0, 0)),
            pl.BlockSpec((1, g), lambda i: (0, 0)),
        ],
        out_specs=(pl.BlockSpec((bm, g), lambda i: (i, 0)),
                   pl.BlockSpec((bm, npad), lambda i: (i, 0))),
        out_shape=(jax.ShapeDtypeStruct((n, g), jnp.bfloat16),
                   jax.ShapeDtypeStruct((n, npad), jnp.bfloat16)),
        compiler_params=pltpu.CompilerParams(
            dimension_semantics=("parallel",)),
    )(adj, s, b, w, c)


def _agg_rest(adj_bf, st_pad, b, w, c, bm, out_dtype):
    n = adj_bf.shape[0]
    npad = adj_bf.shape[1]
    f = st_pad.shape[0]
    g = w.shape[1]
    kern = functools.partial(_agg_rest_kernel, nvalid=n)
    return pl.pallas_call(
        kern,
        grid=(npad // bm,),
        in_specs=[
            pl.BlockSpec((bm, npad), lambda i: (i, 0)),
            pl.BlockSpec((f, npad), lambda i: (0, 0)),
            pl.BlockSpec((f, 1), lambda i: (0, 0)),
            pl.BlockSpec((f, g), lambda i: (0, 0)),
            pl.BlockSpec((g, 1), lambda i: (0, 0)),
        ],
        out_specs=pl.BlockSpec((g, bm), lambda i: (0, i)),
        out_shape=jax.ShapeDtypeStruct((g, npad), out_dtype),
        compiler_params=pltpu.CompilerParams(
            dimension_semantics=("parallel",)),
    )(adj_bf, st_pad, b, w, c)


def _branch(adj, s0, bb1, wn1, z1, bb2c, wn2, z2c, bb3c, wh, bhc, bm1, bm2):
    n = adj.shape[0]
    npad = -(-n // 256) * 256
    t, adj_bf = _agg_first(adj, s0, bb1, wn1, z1, bm1)
    tt = jnp.pad(t.T, ((0, 0), (0, npad - n)))
    tt = _agg_rest(adj_bf, tt, bb2c, wn2, z2c, bm2, jnp.bfloat16)
    return _agg_rest(adj_bf, tt, bb3c, wh, bhc, bm2, jnp.float32)


def kernel(x, adj1, adj2, adj3, adj4, adj5, y, index,
           W1, b1, W2, b2, W3, b3, W4, b4, W5, b5, W6, b6,
           Wd1, bd1, Wd2, bd2, Wd3, bd3):
    n = x.shape[0]
    npad = -(-n // 256) * 256
    bm1 = 200 if n % 200 == 0 else n
    bm2 = 1024 if npad % 1024 == 0 else npad

    s1, s4 = _initial_supports(x, W1, W4)

    b1r, b4r = b1[None, :], b4[None, :]
    b2c, b3c = b2[:, None], b3[:, None]
    b5c, b6c = b5[:, None], b6[:, None]
    z3 = jnp.zeros((1, W2.shape[1]), jnp.float32)
    z4c = jnp.zeros((W3.shape[1], 1), jnp.float32)

    o1 = _branch(adj5, s1, b1r, W2, z3, b2c, W3, z4c, b3c,
                 Wd1, bd1[:, None], bm1, bm2)
    o2 = _branch(adj4, s4, b4r, W5, z3, b5c, W6, z4c, b6c,
                 Wd2, bd2[:, None], bm1, bm2)
    o3 = _branch(adj3, s4, b4r, W5, z3, b5c, W6, z4c, b6c,
                 Wd3, bd3[:, None], bm1, bm2)

    return (o1 + o2 + o3)[:, :n].T


# speedup vs baseline: 1.0055x; 1.0055x over previous
import functools

import jax
import jax.numpy as jnp
from jax.experimental import pallas as pl
from jax.experimental.pallas import tpu as pltpu

_PREC = jax.lax.Precision.HIGHEST
_DN = (((1,), (0,)), ((), ()))


def _proj_kernel(x_ref, w1_ref, w4_ref, o1_ref, o4_ref):
    x = x_ref[...].astype(jnp.bfloat16)
    o1_ref[...] = jax.lax.dot_general(
        x, w1_ref[...].astype(jnp.bfloat16), _DN,
        preferred_element_type=jnp.float32).astype(jnp.bfloat16)
    o4_ref[...] = jax.lax.dot_general(
        x, w4_ref[...].astype(jnp.bfloat16), _DN,
        preferred_element_type=jnp.float32).astype(jnp.bfloat16)


def _initial_supports(x, w1, w4):
    n, _ = x.shape
    f1, f4 = w1.shape[1], w4.shape[1]
    return pl.pallas_call(
        _proj_kernel,
        out_shape=(jax.ShapeDtypeStruct((n, f1), jnp.bfloat16),
                   jax.ShapeDtypeStruct((n, f4), jnp.bfloat16)),
    )(x, w1, w4)


def _epilogue(h, b_ref, w_ref, c_ref, out_dtype):
    h = jnp.maximum(h + b_ref[...], 0.0).astype(jnp.bfloat16)
    o = jax.lax.dot_general(
        h, w_ref[...].astype(jnp.bfloat16), _DN,
        preferred_element_type=jnp.float32) + c_ref[...]
    return o.astype(out_dtype)


def _agg_first_kernel(a_ref, s_ref, b_ref, w_ref, c_ref, o_ref, abf_ref):
    a = a_ref[...].astype(jnp.bfloat16)
    k = a_ref.shape[1]
    pad = abf_ref.shape[1] - k
    if pad:
        a_pad = jnp.concatenate(
            [a, jnp.zeros((a.shape[0], pad), jnp.bfloat16)], axis=1)
    else:
        a_pad = a
    abf_ref[...] = a_pad
    h = jax.lax.dot_general(
        a, s_ref[...], _DN, preferred_element_type=jnp.float32)
    o_ref[...] = _epilogue(h, b_ref, w_ref, c_ref, o_ref.dtype)


def _agg_rest_kernel(a0_ref, a1_ref, st_ref, b_ref, w_ref, c_ref, o_ref, *,
                     nvalid):
    dn = (((1,), (1,)), ((), ()))
    ht = jnp.concatenate(
        [jax.lax.dot_general(st_ref[...], a0_ref[...], dn,
                             preferred_element_type=jnp.float32),
         jax.lax.dot_general(st_ref[...], a1_ref[...], dn,
                             preferred_element_type=jnp.float32)], axis=1)
    ht = jnp.maximum(ht + b_ref[...], 0.0).astype(jnp.bfloat16)
    ot = jax.lax.dot_general(
        w_ref[...].astype(jnp.bfloat16), ht, (((0,), (0,)), ((), ())),
        preferred_element_type=jnp.float32) + c_ref[...]
    g, bm = o_ref.shape
    col = pl.program_id(0) * bm + jax.lax.broadcasted_iota(jnp.int32, (g, bm), 1)
    ot = jnp.where(col < nvalid, ot, 0.0)
    o_ref[...] = ot.astype(o_ref.dtype)


def _agg_first(adj, s, b, w, c, bm):
    n = adj.shape[0]
    npad = -(-n // 256) * 256
    f = s.shape[1]
    g = w.shape[1]
    return pl.pallas_call(
        _agg_first_kernel,
        grid=(n // bm,),
        in_specs=[
            pl.BlockSpec((bm, n), lambda i: (i, 0)),
            pl.BlockSpec((n, f), lambda i: (0, 0)),
            pl.BlockSpec((1, f), lambda i: (0, 0)),
            pl.BlockSpec((f, g), lambda i: (0, 0)),
            pl.BlockSpec((1, g), lambda i: (0, 0)),
        ],
        out_specs=(pl.BlockSpec((bm, g), lambda i: (i, 0)),
                   pl.BlockSpec((bm, npad), lambda i: (i, 0))),
        out_shape=(jax.ShapeDtypeStruct((n, g), jnp.bfloat16),
                   jax.ShapeDtypeStruct((n, npad), jnp.bfloat16)),
        compiler_params=pltpu.CompilerParams(
            dimension_semantics=("parallel",)),
    )(adj, s, b, w, c)


def _agg_rest(adj_bf, st_pad, b, w, c, bm, out_dtype):
    n = adj_bf.shape[0]
    npad = adj_bf.shape[1]
    f = st_pad.shape[0]
    g = w.shape[1]
    kern = functools.partial(_agg_rest_kernel, nvalid=n)
    return pl.pallas_call(
        kern,
        grid=(npad // bm,),
        in_specs=[
            pl.BlockSpec((bm // 2, npad), lambda i: (2 * i, 0)),
            pl.BlockSpec((bm // 2, npad), lambda i: (2 * i + 1, 0)),
            pl.BlockSpec((f, npad), lambda i: (0, 0)),
            pl.BlockSpec((f, 1), lambda i: (0, 0)),
            pl.BlockSpec((f, g), lambda i: (0, 0)),
            pl.BlockSpec((g, 1), lambda i: (0, 0)),
        ],
        out_specs=pl.BlockSpec((g, bm), lambda i: (0, i)),
        out_shape=jax.ShapeDtypeStruct((g, npad), out_dtype),
        compiler_params=pltpu.CompilerParams(
            dimension_semantics=("parallel",)),
    )(adj_bf, adj_bf, st_pad, b, w, c)


def _branch(adj, s0, bb1, wn1, z1, bb2c, wn2, z2c, bb3c, wh, bhc, bm1, bm2):
    n = adj.shape[0]
    npad = -(-n // 256) * 256
    t, adj_bf = _agg_first(adj, s0, bb1, wn1, z1, bm1)
    tt = jnp.pad(t.T, ((0, 0), (0, npad - n)))
    tt = _agg_rest(adj_bf, tt, bb2c, wn2, z2c, bm2, jnp.bfloat16)
    return _agg_rest(adj_bf, tt, bb3c, wh, bhc, bm2, jnp.float32)


def kernel(x, adj1, adj2, adj3, adj4, adj5, y, index,
           W1, b1, W2, b2, W3, b3, W4, b4, W5, b5, W6, b6,
           Wd1, bd1, Wd2, bd2, Wd3, bd3):
    n = x.shape[0]
    npad = -(-n // 256) * 256
    bm1 = 400 if n % 400 == 0 else n
    bm2 = 1024 if npad % 1024 == 0 else npad

    s1, s4 = _initial_supports(x, W1, W4)

    b1r, b4r = b1[None, :], b4[None, :]
    b2c, b3c = b2[:, None], b3[:, None]
    b5c, b6c = b5[:, None], b6[:, None]
    z3 = jnp.zeros((1, W2.shape[1]), jnp.float32)
    z4c = jnp.zeros((W3.shape[1], 1), jnp.float32)

    o1 = _branch(adj5, s1, b1r, W2, z3, b2c, W3, z4c, b3c,
                 Wd1, bd1[:, None], bm1, bm2)
    o2 = _branch(adj4, s4, b4r, W5, z3, b5c, W6, z4c, b6c,
                 Wd2, bd2[:, None], bm1, bm2)
    o3 = _branch(adj3, s4, b4r, W5, z3, b5c, W6, z4c, b6c,
                 Wd3, bd3[:, None], bm1, bm2)

    return (o1 + o2 + o3)[:, :n].T


# speedup vs baseline: 1.0064x; 1.0008x over previous
import functools

import jax
import jax.numpy as jnp
from jax.experimental import pallas as pl
from jax.experimental.pallas import tpu as pltpu

_DN = (((1,), (0,)), ((), ()))


def _proj_kernel(x_ref, w1_ref, w4_ref, o1_ref, o4_ref):
    x = x_ref[...].astype(jnp.bfloat16)
    o1_ref[...] = jax.lax.dot_general(
        x, w1_ref[...].astype(jnp.bfloat16), _DN,
        preferred_element_type=jnp.float32).astype(jnp.bfloat16)
    o4_ref[...] = jax.lax.dot_general(
        x, w4_ref[...].astype(jnp.bfloat16), _DN,
        preferred_element_type=jnp.float32).astype(jnp.bfloat16)


def _initial_supports(x, w1, w4):
    n, _ = x.shape
    f1, f4 = w1.shape[1], w4.shape[1]
    return pl.pallas_call(
        _proj_kernel,
        out_shape=(jax.ShapeDtypeStruct((n, f1), jnp.bfloat16),
                   jax.ShapeDtypeStruct((n, f4), jnp.bfloat16)),
    )(x, w1, w4)


def _epilogue(h, b_ref, w_ref, c_ref, out_dtype):
    h = jnp.maximum(h + b_ref[...], 0.0).astype(jnp.bfloat16)
    o = jax.lax.dot_general(
        h, w_ref[...].astype(jnp.bfloat16), _DN,
        preferred_element_type=jnp.float32) + c_ref[...]
    return o.astype(out_dtype)


def _agg_first_kernel(a_ref, s_ref, b_ref, w_ref, c_ref, o_ref, abf_ref):
    a = a_ref[...].astype(jnp.bfloat16)
    k = a_ref.shape[1]
    pad = abf_ref.shape[1] - k
    if pad:
        a_pad = jnp.concatenate(
            [a, jnp.zeros((a.shape[0], pad), jnp.bfloat16)], axis=1)
    else:
        a_pad = a
    abf_ref[...] = a_pad
    h = jax.lax.dot_general(
        a, s_ref[...], _DN, preferred_element_type=jnp.float32)
    o_ref[...] = _epilogue(h, b_ref, w_ref, c_ref, o_ref.dtype)


def _agg_rest_kernel(a_ref, st_ref, b_ref, w_ref, c_ref, o_ref, *, nvalid):
    ht = jax.lax.dot_general(
        st_ref[...], a_ref[...], (((1,), (1,)), ((), ())),
        preferred_element_type=jnp.float32)
    ht = jnp.maximum(ht + b_ref[...], 0.0).astype(jnp.bfloat16)
    ot = jax.lax.dot_general(
        w_ref[...].astype(jnp.bfloat16), ht, (((0,), (0,)), ((), ())),
        preferred_element_type=jnp.float32) + c_ref[...]
    g, bm = o_ref.shape
    col = pl.program_id(0) * bm + jax.lax.broadcasted_iota(jnp.int32, (g, bm), 1)
    ot = jnp.where(col < nvalid, ot, 0.0)
    o_ref[...] = ot.astype(o_ref.dtype)


def _agg_first(adj, s, b, w, c, bm):
    n = adj.shape[0]
    npad = -(-n // 256) * 256
    f = s.shape[1]
    g = w.shape[1]
    return pl.pallas_call(
        _agg_first_kernel,
        grid=(n // bm,),
        in_specs=[
            pl.BlockSpec((bm, n), lambda i: (i, 0)),
            pl.BlockSpec((n, f), lambda i: (0, 0)),
            pl.BlockSpec((1, f), lambda i: (0, 0)),
            pl.BlockSpec((f, g), lambda i: (0, 0)),
            pl.BlockSpec((1, g), lambda i: (0, 0)),
        ],
        out_specs=(pl.BlockSpec((bm, g), lambda i: (i, 0)),
                   pl.BlockSpec((bm, npad), lambda i: (i, 0))),
        out_shape=(jax.ShapeDtypeStruct((n, g), jnp.bfloat16),
                   jax.ShapeDtypeStruct((n, npad), jnp.bfloat16)),
        compiler_params=pltpu.CompilerParams(
            dimension_semantics=("parallel",)),
    )(adj, s, b, w, c)


def _agg_rest(adj_bf, st_pad, b, w, c, bm, out_dtype):
    n = adj_bf.shape[0]
    npad = adj_bf.shape[1]
    f = st_pad.shape[0]
    g = w.shape[1]
    kern = functools.partial(_agg_rest_kernel, nvalid=n)
    return pl.pallas_call(
        kern,
        grid=(npad // bm,),
        in_specs=[
            pl.BlockSpec((bm, npad), lambda i: (i, 0)),
            pl.BlockSpec((f, npad), lambda i: (0, 0)),
            pl.BlockSpec((f, 1), lambda i: (0, 0)),
            pl.BlockSpec((f, g), lambda i: (0, 0)),
            pl.BlockSpec((g, 1), lambda i: (0, 0)),
        ],
        out_specs=pl.BlockSpec((g, bm), lambda i: (0, i)),
        out_shape=jax.ShapeDtypeStruct((g, npad), out_dtype),
        compiler_params=pltpu.CompilerParams(
            dimension_semantics=("parallel",)),
    )(adj_bf, st_pad, b, w, c)


def _branch(adj, s0, bb1, wn1, z1, bb2c, wn2, z2c, bb3c, wh, bhc, bm1, bm2):
    n = adj.shape[0]
    npad = -(-n // 256) * 256
    t, adj_bf = _agg_first(adj, s0, bb1, wn1, z1, bm1)
    tt = jnp.pad(t.T, ((0, 0), (0, npad - n)))
    tt = _agg_rest(adj_bf, tt, bb2c, wn2, z2c, bm2, jnp.bfloat16)
    return _agg_rest(adj_bf, tt, bb3c, wh, bhc, bm2, jnp.float32)


def kernel(x, adj1, adj2, adj3, adj4, adj5, y, index,
           W1, b1, W2, b2, W3, b3, W4, b4, W5, b5, W6, b6,
           Wd1, bd1, Wd2, bd2, Wd3, bd3):
    n = x.shape[0]
    npad = -(-n // 256) * 256
    bm1 = 400 if n % 400 == 0 else n
    bm2 = 1024 if npad % 1024 == 0 else npad

    s1, s4 = _initial_supports(x, W1, W4)

    b1r, b4r = b1[None, :], b4[None, :]
    b2c, b3c = b2[:, None], b3[:, None]
    b5c, b6c = b5[:, None], b6[:, None]
    z3 = jnp.zeros((1, W2.shape[1]), jnp.float32)
    z4c = jnp.zeros((W3.shape[1], 1), jnp.float32)

    o1 = _branch(adj5, s1, b1r, W2, z3, b2c, W3, z4c, b3c,
                 Wd1, bd1[:, None], bm1, bm2)
    o2 = _branch(adj4, s4, b4r, W5, z3, b5c, W6, z4c, b6c,
                 Wd2, bd2[:, None], bm1, bm2)
    o3 = _branch(adj3, s4, b4r, W5, z3, b5c, W6, z4c, b6c,
                 Wd3, bd3[:, None], bm1, bm2)

    return (o1 + o2 + o3)[:, :n].T
